# baseline (speedup 1.0000x reference)
import functools

import jax
import jax.numpy as jnp
from jax.experimental import pallas as pl
from jax.experimental.pallas import tpu as pltpu

_NUM_EXPERTS = 8
_D = 768
_HEADS = 12
_DEPTH = 64
_LANE = 128
_BT = 256
_NEG = -1e30


def _fused_body(x_ref, xb_ref, wr_ref, br_ref, we_ref, be_ref, wq_ref, bq_ref,
                wk_ref, bk_ref, wv_ref, bv_ref, wo_ref, bo_ref,
                hmap_ref, hmapt_ref, o_ref):
    x = x_ref[...]

    logits = jnp.dot(x, wr_ref[...], preferred_element_type=jnp.float32)
    logits = logits + br_ref[...]
    m = jnp.max(logits, axis=-1, keepdims=True)
    e = jnp.exp(logits - m)
    probs = e / jnp.sum(e, axis=-1, keepdims=True)

    cols = jax.lax.broadcasted_iota(jnp.int32, probs.shape, 1)
    p1 = jnp.max(probs, axis=-1, keepdims=True)
    i1 = jnp.min(jnp.where(probs >= p1, cols, _LANE), axis=-1, keepdims=True)
    probs_m = jnp.where(cols == i1, -1.0, probs)
    p2 = jnp.max(probs_m, axis=-1, keepdims=True)
    i2 = jnp.min(jnp.where(probs_m >= p2, cols, _LANE), axis=-1, keepdims=True)
    sel = (cols == i1) | (cols == i2)
    w = jnp.where(sel, probs, 0.0)

    xb = xb_ref[...]
    combined = jnp.zeros((x.shape[0], _D), dtype=jnp.float32)
    for i in range(_NUM_EXPERTS):
        eo = jnp.dot(xb, we_ref[i], preferred_element_type=jnp.float32)
        eo = jnp.maximum(eo + be_ref[i:i + 1, :], 0.0)
        combined = combined + eo * w[:, i:i + 1]

    cb = combined.astype(jnp.bfloat16)
    q = jnp.dot(cb, wq_ref[...], preferred_element_type=jnp.float32) + bq_ref[...]
    k = jnp.dot(cb, wk_ref[...], preferred_element_type=jnp.float32) + bk_ref[...]
    v = jnp.dot(cb, wv_ref[...], preferred_element_type=jnp.float32) + bv_ref[...]
    hmap = hmap_ref[...]
    s = jnp.dot((q * k).astype(jnp.bfloat16), hmap,
                preferred_element_type=jnp.float32)
    s = s * (1.0 / jnp.sqrt(jnp.float32(_DEPTH)))
    s = jnp.where(jax.lax.broadcasted_iota(jnp.int32, s.shape, 1) < _HEADS,
                  s, _NEG)
    sm = jnp.max(s, axis=-1, keepdims=True)
    se = jnp.exp(s - sm)
    aw = se / jnp.sum(se, axis=-1, keepdims=True)
    wb = jnp.dot(aw.astype(jnp.bfloat16), hmapt_ref[...],
                 preferred_element_type=jnp.float32)
    out = jnp.dot((wb * v).astype(jnp.bfloat16), wo_ref[...],
                  preferred_element_type=jnp.float32)
    o_ref[...] = out + bo_ref[...]


@jax.jit
def kernel(inputs, Wr, br, We, be, Wq, bq, Wk, bk, Wv, bv, Wo, bo):
    n = inputs.shape[0]
    wr_p = jnp.zeros((_D, _LANE), jnp.float32).at[:, :_NUM_EXPERTS].set(Wr)
    br_p = jnp.full((1, _LANE), _NEG, jnp.float32).at[0, :_NUM_EXPERTS].set(br)
    d_idx = jnp.arange(_D) // _DEPTH
    hmap = (d_idx[:, None] == jnp.arange(_LANE)[None, :]).astype(jnp.bfloat16)
    hmapt = hmap.T

    bf = jnp.bfloat16
    xb = inputs.astype(bf)
    grid = (n // _BT,)
    full = lambda shape: pl.BlockSpec(shape, lambda i: (0,) * len(shape))
    out = pl.pallas_call(
        _fused_body,
        grid=grid,
        in_specs=[
            pl.BlockSpec((_BT, _D), lambda i: (i, 0)),
            pl.BlockSpec((_BT, _D), lambda i: (i, 0)),
            full((_D, _LANE)),
            full((1, _LANE)),
            full((_NUM_EXPERTS, _D, _D)),
            full((_NUM_EXPERTS, _D)),
            full((_D, _D)), full((1, _D)),
            full((_D, _D)), full((1, _D)),
            full((_D, _D)), full((1, _D)),
            full((_D, _D)), full((1, _D)),
            full((_D, _LANE)),
            full((_LANE, _D)),
        ],
        out_specs=pl.BlockSpec((_BT, _D), lambda i: (i, 0)),
        out_shape=jax.ShapeDtypeStruct((n, _D), jnp.float32),
        compiler_params=pltpu.CompilerParams(
            dimension_semantics=("arbitrary",),
        ),
    )(inputs, xb, wr_p, br_p, We.astype(bf), be,
      Wq.astype(bf), bq.reshape(1, _D), Wk.astype(bf), bk.reshape(1, _D),
      Wv.astype(bf), bv.reshape(1, _D), Wo.astype(bf), bo.reshape(1, _D),
      hmap, hmapt)
    return out


# speedup vs baseline: 1.1717x; 1.1717x over previous
import jax
import jax.numpy as jnp
from jax.experimental import pallas as pl
from jax.experimental.pallas import tpu as pltpu

_NUM_EXPERTS = 8
_D = 768
_HEADS = 12
_DEPTH = 64
_LANE = 128
_BT = 512
_NEG = -1e30


def _cast_body(we_ref, wq_ref, wk_ref, wv_ref, wo_ref,
               web_ref, wqb_ref, wkb_ref, wvb_ref, wob_ref):
    i = pl.program_id(0)
    web_ref[...] = we_ref[...].astype(jnp.bfloat16)

    @pl.when(i == 0)
    def _():
        wqb_ref[...] = wq_ref[...].astype(jnp.bfloat16)
        wkb_ref[...] = wk_ref[...].astype(jnp.bfloat16)
        wvb_ref[...] = wv_ref[...].astype(jnp.bfloat16)
        wob_ref[...] = wo_ref[...].astype(jnp.bfloat16)


def _fused_body(x_ref, wr_ref, br_ref, we_ref, be_ref, wq_ref, bq_ref,
                wk_ref, bk_ref, wv_ref, bv_ref, wo_ref, bo_ref,
                hmap_ref, hmapt_ref, o_ref):
    x = x_ref[...]

    logits = jnp.dot(x, wr_ref[...], preferred_element_type=jnp.float32)
    logits = logits + br_ref[...]
    m = jnp.max(logits, axis=-1, keepdims=True)
    e = jnp.exp(logits - m)
    probs = e / jnp.sum(e, axis=-1, keepdims=True)

    cols = jax.lax.broadcasted_iota(jnp.int32, probs.shape, 1)
    p1 = jnp.max(probs, axis=-1, keepdims=True)
    i1 = jnp.min(jnp.where(probs >= p1, cols, _LANE), axis=-1, keepdims=True)
    probs_m = jnp.where(cols == i1, -1.0, probs)
    p2 = jnp.max(probs_m, axis=-1, keepdims=True)
    i2 = jnp.min(jnp.where(probs_m >= p2, cols, _LANE), axis=-1, keepdims=True)
    sel = (cols == i1) | (cols == i2)
    w = jnp.where(sel, probs, 0.0)

    xb = x.astype(jnp.bfloat16)
    combined = jnp.zeros((x.shape[0], _D), dtype=jnp.float32)
    for i in range(_NUM_EXPERTS):
        eo = jnp.dot(xb, we_ref[i], preferred_element_type=jnp.float32)
        eo = jnp.maximum(eo + be_ref[i:i + 1, :], 0.0)
        combined = combined + eo * w[:, i:i + 1]

    cb = combined.astype(jnp.bfloat16)
    q = jnp.dot(cb, wq_ref[...], preferred_element_type=jnp.float32) + bq_ref[...]
    k = jnp.dot(cb, wk_ref[...], preferred_element_type=jnp.float32) + bk_ref[...]
    v = jnp.dot(cb, wv_ref[...], preferred_element_type=jnp.float32) + bv_ref[...]
    hmap = hmap_ref[...]
    s = jnp.dot((q * k).astype(jnp.bfloat16), hmap,
                preferred_element_type=jnp.float32)
    s = s * (1.0 / jnp.sqrt(jnp.float32(_DEPTH)))
    s = jnp.where(jax.lax.broadcasted_iota(jnp.int32, s.shape, 1) < _HEADS,
                  s, _NEG)
    sm = jnp.max(s, axis=-1, keepdims=True)
    se = jnp.exp(s - sm)
    aw = se / jnp.sum(se, axis=-1, keepdims=True)
    wb = jnp.dot(aw.astype(jnp.bfloat16), hmapt_ref[...],
                 preferred_element_type=jnp.float32)
    out = jnp.dot((wb * v).astype(jnp.bfloat16), wo_ref[...],
                  preferred_element_type=jnp.float32)
    o_ref[...] = out + bo_ref[...]


@jax.jit
def kernel(inputs, Wr, br, We, be, Wq, bq, Wk, bk, Wv, bv, Wo, bo):
    n = inputs.shape[0]
    bf = jnp.bfloat16

    dxd = lambda dt: pl.BlockSpec((_D, _D), lambda i: (0, 0))
    web, wqb, wkb, wvb, wob = pl.pallas_call(
        _cast_body,
        grid=(_NUM_EXPERTS,),
        in_specs=[pl.BlockSpec((1, _D, _D), lambda i: (i, 0, 0)),
                  dxd(None), dxd(None), dxd(None), dxd(None)],
        out_specs=[pl.BlockSpec((1, _D, _D), lambda i: (i, 0, 0)),
                   dxd(None), dxd(None), dxd(None), dxd(None)],
        out_shape=[jax.ShapeDtypeStruct((_NUM_EXPERTS, _D, _D), bf),
                   jax.ShapeDtypeStruct((_D, _D), bf),
                   jax.ShapeDtypeStruct((_D, _D), bf),
                   jax.ShapeDtypeStruct((_D, _D), bf),
                   jax.ShapeDtypeStruct((_D, _D), bf)],
        compiler_params=pltpu.CompilerParams(
            dimension_semantics=("arbitrary",),
        ),
    )(We, Wq, Wk, Wv, Wo)

    wr_p = jnp.zeros((_D, _LANE), jnp.float32).at[:, :_NUM_EXPERTS].set(Wr)
    br_p = jnp.full((1, _LANE), _NEG, jnp.float32).at[0, :_NUM_EXPERTS].set(br)
    d_idx = jnp.arange(_D) // _DEPTH
    hmap = (d_idx[:, None] == jnp.arange(_LANE)[None, :]).astype(bf)
    hmapt = hmap.T

    grid = (n // _BT,)
    full = lambda shape: pl.BlockSpec(shape, lambda i: (0,) * len(shape))
    out = pl.pallas_call(
        _fused_body,
        grid=grid,
        in_specs=[
            pl.BlockSpec((_BT, _D), lambda i: (i, 0)),
            full((_D, _LANE)),
            full((1, _LANE)),
            full((_NUM_EXPERTS, _D, _D)),
            full((_NUM_EXPERTS, _D)),
            full((_D, _D)), full((1, _D)),
            full((_D, _D)), full((1, _D)),
            full((_D, _D)), full((1, _D)),
            full((_D, _D)), full((1, _D)),
            full((_D, _LANE)),
            full((_LANE, _D)),
        ],
        out_specs=pl.BlockSpec((_BT, _D), lambda i: (i, 0)),
        out_shape=jax.ShapeDtypeStruct((n, _D), jnp.float32),
        compiler_params=pltpu.CompilerParams(
            dimension_semantics=("arbitrary",),
        ),
    )(inputs, wr_p, br_p, web, be,
      wqb, bq.reshape(1, _D), wkb, bk.reshape(1, _D),
      wvb, bv.reshape(1, _D), wob, bo.reshape(1, _D),
      hmap, hmapt)
    return out
